# baseline (speedup 1.0000x reference)
import functools

import jax
import jax.numpy as jnp
from jax import lax
from jax.experimental import pallas as pl
from jax.experimental.pallas import tpu as pltpu
from jax.experimental.pallas import tpu_sc as plsc

N = 10000
E = 160000
IN = 128
HID = 128
OUT = 2048

NSC = 2
NT = 16
NPAD = 10240
ROWS_T = NPAD // NT
EDGE_SC = E // NSC
EDGE_T = EDGE_SC // NT
EDGE_PAD_T = 5120
E_PAD = EDGE_PAD_T * NSC * NT
SEG_B = 256
SEG_FULL = EDGE_PAD_T // SEG_B

EDGE_ALL_T = E // NT

DEG_B = 256
DEG_FULL = EDGE_PAD_T // DEG_B

BROWS = 512
NBLOCK_SC = 10
BEL = BROWS * OUT
BPAD = BEL
ZCH = 32768
SCHUNK = 4096
FTILE = BEL // NT
PPAD = 20480
SENT = 1 << 30

BLK = 400
NBLK = N // BLK

@functools.cache
def _mesh():
    return plsc.VectorSubcoreMesh(core_axis_name="c", subcore_axis_name="s")


def _seg_body(g_hbm, src_hbm, dst_hbm, zeros_hbm, out_hbm,
              idx_s, idx_d, rows, acc, sem):
    c = lax.axis_index("c")
    s = lax.axis_index("s")
    pltpu.sync_copy(zeros_hbm, acc.at[pl.ds(s * ROWS_T, ROWS_T)])
    plsc.subcore_barrier()

    base = (c * NT + s) * EDGE_PAD_T

    def batch(k, _):
        off = base + k * SEG_B
        pltpu.sync_copy(src_hbm.at[pl.ds(off, SEG_B)], idx_s)
        pltpu.sync_copy(dst_hbm.at[pl.ds(off, SEG_B)], idx_d)
        pltpu.async_copy(g_hbm.at[idx_s], rows, sem).wait()
        pltpu.sync_copy(rows, acc.at[idx_d], add=True)
        return _

    lax.fori_loop(0, SEG_FULL, batch, None)

    plsc.subcore_barrier()
    pltpu.sync_copy(acc.at[pl.ds(s * ROWS_T, ROWS_T)],
                    out_hbm.at[c, pl.ds(s * ROWS_T, ROWS_T)])


@functools.cache
def _seg_kernel():
  return pl.kernel(
    _seg_body,
    out_type=jax.ShapeDtypeStruct((NSC, NPAD, HID), jnp.float32),
    mesh=_mesh(),
    scratch_types=[
        pltpu.VMEM((SEG_B,), jnp.int32),
        pltpu.VMEM((SEG_B,), jnp.int32),
        pltpu.VMEM((SEG_B, HID), jnp.float32),
        pltpu.VMEM_SHARED((NPAD + 8, HID), jnp.float32),
        pltpu.SemaphoreType.DMA,
    ],
  )


def _seg_call(*args):
    return _seg_kernel()(*args)


def _mask_body(src_hbm, dst_hbm, zeros_hbm, ones_hbm, mask_hbm,
               pall, prel, zero_v, ones_v, mblk):
    c = lax.axis_index("c")
    s = lax.axis_index("s")

    pltpu.sync_copy(zeros_hbm, zero_v)
    pltpu.sync_copy(ones_hbm, ones_v)
    pltpu.sync_copy(src_hbm.at[pl.ds(s * EDGE_ALL_T, EDGE_ALL_T)],
                    pall.at[pl.ds(0, EDGE_ALL_T)])
    pltpu.sync_copy(dst_hbm.at[pl.ds(s * EDGE_ALL_T, EDGE_ALL_T)],
                    pall.at[pl.ds(PPAD // 2, EDGE_ALL_T)])

    def pos(k, _):
        for u in range(5):
            o = (k * 5 + u) * 16
            sv = pall[pl.ds(o, 16)]
            dv = pall[pl.ds(PPAD // 2 + o, 16)]
            pall[pl.ds(o, 16)] = jnp.where(sv < OUT, dv * OUT + sv, SENT)
            pall[pl.ds(PPAD // 2 + o, 16)] = jnp.where(dv < OUT,
                                                       sv * OUT + dv, SENT)
        return _

    lax.fori_loop(0, EDGE_ALL_T // 80, pos, None)
    for k in range(EDGE_ALL_T // 16, PPAD // 2 // 16):
        pall[pl.ds(k * 16, 16)] = jnp.full((16,), SENT, jnp.int32)
        pall[pl.ds(PPAD // 2 + k * 16, 16)] = jnp.full((16,), SENT, jnp.int32)

    def block(b, _):
        base = (NBLOCK_SC * c + b) * BEL

        for z in range(FTILE // ZCH):
            pltpu.sync_copy(zero_v, mblk.at[pl.ds(s * FTILE + z * ZCH, ZCH)])

        @pl.when(s == 0)
        def _pad():
            pltpu.sync_copy(zero_v.at[pl.ds(0, 8)], mblk.at[pl.ds(BPAD, 8)])

        plsc.subcore_barrier()

        def sb(kb, _):
            def sub(j, _):
                for u in range(8):
                    o = (j * 8 + u) * 16
                    pa = pall[pl.ds(kb * SCHUNK + o, 16)]
                    pr = pa - base
                    ok = (pr >= 0) & (pr < BEL)
                    prel[pl.ds(o, 16)] = jnp.where(ok, pr, BPAD)
                return _

            lax.fori_loop(0, SCHUNK // 128, sub, None)
            pltpu.sync_copy(ones_v, mblk.at[prel])
            return _

        lax.fori_loop(0, PPAD // SCHUNK, sb, None)
        plsc.subcore_barrier()

        pltpu.sync_copy(mblk.at[pl.ds(s * FTILE, FTILE)],
                        mask_hbm.at[pl.ds(base + s * FTILE, FTILE)])
        return _

    lax.fori_loop(0, NBLOCK_SC, block, None)


@functools.cache
def _mask_kernel():
  return pl.kernel(
    _mask_body,
    out_type=jax.ShapeDtypeStruct((NPAD * OUT,), jnp.float32),
    mesh=_mesh(),
    scratch_types=[
        pltpu.VMEM((PPAD,), jnp.int32),
        pltpu.VMEM((SCHUNK,), jnp.int32),
        pltpu.VMEM((ZCH,), jnp.float32),
        pltpu.VMEM((SCHUNK,), jnp.float32),
        pltpu.VMEM_SHARED((BEL + 8,), jnp.float32),
    ],
  )


def _mask_call(*args):
    return _mask_kernel()(*args)


def _deg_body(src_hbm, dst_hbm, zeros_hbm, ones_hbm, deg_hbm,
              idx_d, ones_v, acc):
    c = lax.axis_index("c")
    s = lax.axis_index("s")
    pltpu.sync_copy(ones_hbm, ones_v)
    pltpu.sync_copy(zeros_hbm, acc.at[pl.ds(s * ROWS_T, ROWS_T)])
    plsc.subcore_barrier()

    base = (c * NT + s) * EDGE_PAD_T

    def batch(k, _):
        off = base + k * DEG_B
        pltpu.sync_copy(dst_hbm.at[pl.ds(off, DEG_B)], idx_d)
        pltpu.sync_copy(ones_v, acc.at[idx_d], add=True)
        return _

    lax.fori_loop(0, DEG_FULL, batch, None)

    plsc.subcore_barrier()
    pltpu.sync_copy(acc.at[pl.ds(s * ROWS_T, ROWS_T)],
                    deg_hbm.at[c, pl.ds(s * ROWS_T, ROWS_T)])


@functools.cache
def _deg_kernel():
  return pl.kernel(
    _deg_body,
    out_type=jax.ShapeDtypeStruct((NSC, NPAD, HID), jnp.float32),
    mesh=_mesh(),
    scratch_types=[
        pltpu.VMEM((DEG_B,), jnp.int32),
        pltpu.VMEM((DEG_B, HID), jnp.float32),
        pltpu.VMEM_SHARED((NPAD + 8, HID), jnp.float32),
    ],
  )


def _deg_call(*args):
    return _deg_kernel()(*args)


def _dot(a, b):
    return jnp.dot(a, b, preferred_element_type=jnp.float32)


def _tc_a_body(x, sfm, tfm, og, op, wl, wr, g1, r1):
    parts = (x[...], sfm[...], tfm[...], og[...], op[...])
    wlv = wl[...]
    wrv = wr[...]
    g = _dot(parts[0], wlv[0:IN])
    r = _dot(parts[0], wrv[0:IN])
    for k in range(1, 5):
        g = g + _dot(parts[k], wlv[k * IN:(k + 1) * IN])
        r = r + _dot(parts[k], wrv[k * IN:(k + 1) * IN])
    g1[...] = g
    r1[...] = r


def _tc_a(x, sfm, tfm, og, op, wl1t, wr1t):
    bs = pl.BlockSpec((BLK, IN), lambda i: (i, 0))
    ws = pl.BlockSpec((5 * IN, HID), lambda i: (0, 0))
    return pl.pallas_call(
        _tc_a_body,
        grid=(NBLK,),
        in_specs=[bs, bs, bs, bs, bs, ws, ws],
        out_specs=[pl.BlockSpec((BLK, HID), lambda i: (i, 0))] * 2,
        out_shape=[jax.ShapeDtypeStruct((N, HID), jnp.float32)] * 2,
    )(x, sfm, tfm, og, op, wl1t, wr1t)


def _deg_stats(degp):
    deg = degp[0, :, 0] + degp[1, :, 0]
    invdeg = 1.0 / jnp.maximum(deg, 1.0)
    degpos = (deg > 0).astype(jnp.float32)
    return invdeg, degpos


def _layernorm(h, g, b):
    m = jnp.mean(h, axis=-1, keepdims=True)
    v = jnp.var(h, axis=-1, keepdims=True)
    return (h - m) / jnp.sqrt(v + 1e-5) * g + b


def _tc_b_body(s1, degp, r1, tfm, bl1, g1n, b1n, wl2a, wr2a, g2, r2, t):
    i = pl.program_id(0)
    invdeg, _ = _deg_stats(degp[...])
    pre = (s1[0] + s1[1]) * invdeg[:, None] + bl1[...] + r1[...]
    h1 = _layernorm(jax.nn.relu(pre), g1n[...], b1n[...])
    g2[...] = _dot(h1, wl2a[...])
    r2[...] = _dot(h1, wr2a[...])
    flag = jnp.any(tfm[...] != 0, axis=1).astype(jnp.float32)
    tp = _dot(flag[None, :], h1)

    @pl.when(i == 0)
    def _init():
        t[...] = jnp.zeros_like(t)

    t[...] += tp


def _tc_b(s1, degp, r1, tfm, bl1, g1n, b1n, wl2at, wr2at):
    bs = pl.BlockSpec((BLK, HID), lambda i: (i, 0))
    ws = pl.BlockSpec((HID, HID), lambda i: (0, 0))
    vs = pl.BlockSpec((1, HID), lambda i: (0, 0))
    return pl.pallas_call(
        _tc_b_body,
        grid=(NBLK,),
        in_specs=[pl.BlockSpec((NSC, BLK, HID), lambda i: (0, i, 0)),
                  pl.BlockSpec((NSC, BLK, HID), lambda i: (0, i, 0)),
                  bs, bs, vs, vs, vs, ws, ws],
        out_specs=[bs, bs, vs],
        out_shape=[jax.ShapeDtypeStruct((N, HID), jnp.float32),
                   jax.ShapeDtypeStruct((N, HID), jnp.float32),
                   jax.ShapeDtypeStruct((1, HID), jnp.float32)],
    )(s1, degp, r1, tfm, bl1, g1n, b1n, wl2at, wr2at)


def _tc_c_body(s2, degp, r2, t, wl2b, wr2b, bl2, g2n, b2n, wl3, wr3, g3, r3):
    invdeg, degpos = _deg_stats(degp[...])
    tv = t[...]
    tl = _dot(tv, wl2b[...])
    tr = _dot(tv, wr2b[...])
    pre = ((s2[0] + s2[1]) * invdeg[:, None] + degpos[:, None] * tl
           + bl2[...] + r2[...] + tr)
    h2 = _layernorm(jax.nn.relu(pre), g2n[...], b2n[...])
    g3[...] = _dot(h2, wl3[...])
    r3[...] = _dot(h2, wr3[...])


def _tc_c(s2, degp, r2, t, wl2bt, wr2bt, bl2, g2n, b2n, wl3t, wr3t):
    bs = pl.BlockSpec((BLK, HID), lambda i: (i, 0))
    ws = pl.BlockSpec((HID, HID), lambda i: (0, 0))
    vs = pl.BlockSpec((1, HID), lambda i: (0, 0))
    return pl.pallas_call(
        _tc_c_body,
        grid=(NBLK,),
        in_specs=[pl.BlockSpec((NSC, BLK, HID), lambda i: (0, i, 0)),
                  pl.BlockSpec((NSC, BLK, HID), lambda i: (0, i, 0)),
                  bs, vs, ws, ws, vs, vs, vs, ws, ws],
        out_specs=[bs, bs],
        out_shape=[jax.ShapeDtypeStruct((N, HID), jnp.float32)] * 2,
    )(s2, degp, r2, t, wl2bt, wr2bt, bl2, g2n, b2n, wl3t, wr3t)


def _tc_d_body(s3, degp, r3, bl3, fcw, fcb, mask, out):
    i = pl.program_id(0)
    invdeg, _ = _deg_stats(degp[...])
    h3 = jax.nn.relu((s3[0] + s3[1]) * invdeg[:, None] + bl3[...] + r3[...])
    logits = _dot(h3, fcw[...]) + fcb[...]
    rid = i * BLK + lax.broadcasted_iota(jnp.int32, (BLK, OUT), 0)
    cid = lax.broadcasted_iota(jnp.int32, (BLK, OUT), 1)
    maskv = jnp.maximum(mask[...], (rid == cid).astype(jnp.float32))
    mx = jnp.max(logits, axis=1, keepdims=True)
    e = jnp.exp(logits - mx) * maskv
    z = jnp.sum(e, axis=1, keepdims=True)
    out[...] = jnp.where(z > 0, e / jnp.where(z > 0, z, 1.0), 0.0)


def _tc_d(s3, degp, r3, bl3, fcwt, fcb, maskm):
    bs = pl.BlockSpec((BLK, HID), lambda i: (i, 0))
    return pl.pallas_call(
        _tc_d_body,
        grid=(NBLK,),
        in_specs=[pl.BlockSpec((NSC, BLK, HID), lambda i: (0, i, 0)),
                  pl.BlockSpec((NSC, BLK, HID), lambda i: (0, i, 0)),
                  bs,
                  pl.BlockSpec((1, HID), lambda i: (0, 0)),
                  pl.BlockSpec((HID, OUT), lambda i: (0, 0)),
                  pl.BlockSpec((1, OUT), lambda i: (0, 0)),
                  pl.BlockSpec((BLK, OUT), lambda i: (i, 0))],
        out_specs=pl.BlockSpec((BLK, OUT), lambda i: (i, 0)),
        out_shape=jax.ShapeDtypeStruct((N, OUT), jnp.float32),
    )(s3, degp, r3, bl3, fcwt, fcb, maskm)


def kernel(x, start_feature_masked, target_feature_masked, other_goals,
           other_pos, edge_index,
           conv1_Wl, conv1_bl, conv1_Wr, conv2_Wl, conv2_bl, conv2_Wr,
           conv3_Wl, conv3_bl, conv3_Wr, fc_W, fc_b,
           ln1_g, ln1_b, ln2_g, ln2_b):
    src = edge_index[0]
    dst = edge_index[1]

    wl1t = conv1_Wl.T
    wr1t = conv1_Wr.T
    wl2at = conv2_Wl[:, :HID].T
    wl2bt = conv2_Wl[:, HID:].T
    wr2at = conv2_Wr[:, :HID].T
    wr2bt = conv2_Wr[:, HID:].T
    wl3t = conv3_Wl.T
    wr3t = conv3_Wr.T
    fcwt = fc_W.T

    bl1 = conv1_bl.reshape(1, HID)
    bl2 = conv2_bl.reshape(1, HID)
    bl3 = conv3_bl.reshape(1, HID)
    fcb = fc_b.reshape(1, OUT)
    g1n = ln1_g.reshape(1, HID)
    b1n = ln1_b.reshape(1, HID)
    g2n = ln2_g.reshape(1, HID)
    b2n = ln2_b.reshape(1, HID)

    pad_s = jnp.zeros((E_PAD - E,), jnp.int32)
    pad_d = jnp.full((E_PAD - E,), NPAD, jnp.int32)
    src_p = jnp.concatenate([src, pad_s])
    dst_p = jnp.concatenate([dst, pad_d])

    zeros_a = jnp.zeros((ROWS_T, HID), jnp.float32)
    zeros_m = jnp.zeros((ZCH,), jnp.float32)
    ones_d = jnp.ones((DEG_B, HID), jnp.float32)
    ones_m = jnp.ones((SCHUNK,), jnp.float32)

    mask_flat = _mask_call(src, dst, zeros_m, ones_m)
    degp = _deg_call(src_p, dst_p, zeros_a, ones_d)
    maskm = mask_flat.reshape(NPAD, OUT)

    g1, r1 = _tc_a(x, start_feature_masked, target_feature_masked,
                   other_goals, other_pos, wl1t, wr1t)
    s1 = _seg_call(g1, src_p, dst_p, zeros_a)
    g2, r2, t = _tc_b(s1, degp, r1, target_feature_masked, bl1, g1n, b1n,
                      wl2at, wr2at)
    s2 = _seg_call(g2, src_p, dst_p, zeros_a)
    g3, r3 = _tc_c(s2, degp, r2, t, wl2bt, wr2bt, bl2, g2n, b2n, wl3t, wr3t)
    s3 = _seg_call(g3, src_p, dst_p, zeros_a)
    return _tc_d(s3, degp, r3, bl3, fcwt, fcb, maskm)


# speedup vs baseline: 1.4313x; 1.4313x over previous
import functools

import jax
import jax.numpy as jnp
from jax import lax
from jax.experimental import pallas as pl
from jax.experimental.pallas import tpu as pltpu
from jax.experimental.pallas import tpu_sc as plsc

N = 10000
E = 160000
IN = 128
HID = 128
OUT = 2048

NSC = 2
NT = 16
NPAD = 10240
ROWS_T = NPAD // NT
EDGE_SC = E // NSC
EDGE_T = EDGE_SC // NT
SEG_B = 200
SEG_FULL = EDGE_T // SEG_B

EDGE_ALL_T = E // NT

DEG_B = 200
DEG_FULL = EDGE_T // DEG_B

BROWS = 512
NBLOCK_SC = 10
BEL = BROWS * OUT
BPAD = BEL
ZCH = 32768
SCHUNK = 4096
FTILE = BEL // NT
PPAD = 20480
SENT = 1 << 30

BLK = 400
NBLK = N // BLK

@functools.cache
def _mesh():
    return plsc.VectorSubcoreMesh(core_axis_name="c", subcore_axis_name="s")


def _seg_body(g_hbm, src_hbm, dst_hbm, zeros_hbm, out_hbm,
              idx_s, idx_d, rows, acc, sem):
    c = lax.axis_index("c")
    s = lax.axis_index("s")
    pltpu.sync_copy(zeros_hbm, acc.at[pl.ds(s * ROWS_T, ROWS_T)])
    plsc.subcore_barrier()

    base = c * EDGE_SC + s * EDGE_T

    def batch(k, _):
        off = base + k * SEG_B
        pltpu.sync_copy(src_hbm.at[pl.ds(off, SEG_B)], idx_s)
        pltpu.sync_copy(dst_hbm.at[pl.ds(off, SEG_B)], idx_d)
        pltpu.async_copy(g_hbm.at[idx_s], rows, sem).wait()
        pltpu.sync_copy(rows, acc.at[idx_d], add=True)
        return _

    lax.fori_loop(0, SEG_FULL, batch, None)

    plsc.subcore_barrier()
    pltpu.sync_copy(acc.at[pl.ds(s * ROWS_T, ROWS_T)],
                    out_hbm.at[c, pl.ds(s * ROWS_T, ROWS_T)])


@functools.cache
def _seg_kernel():
  return pl.kernel(
    _seg_body,
    out_type=jax.ShapeDtypeStruct((NSC, NPAD, HID), jnp.float32),
    mesh=_mesh(),
    scratch_types=[
        pltpu.VMEM((SEG_B,), jnp.int32),
        pltpu.VMEM((SEG_B,), jnp.int32),
        pltpu.VMEM((SEG_B, HID), jnp.float32),
        pltpu.VMEM_SHARED((NPAD, HID), jnp.float32),
        pltpu.SemaphoreType.DMA,
    ],
  )


def _seg_call(*args):
    return _seg_kernel()(*args)


def _mask_body(src_hbm, dst_hbm, zeros_hbm, ones_hbm, mask_hbm,
               pall, prel, zero_v, ones_v, mblk):
    c = lax.axis_index("c")
    s = lax.axis_index("s")

    pltpu.sync_copy(zeros_hbm, zero_v)
    pltpu.sync_copy(ones_hbm, ones_v)
    pltpu.sync_copy(src_hbm.at[pl.ds(s * EDGE_ALL_T, EDGE_ALL_T)],
                    pall.at[pl.ds(0, EDGE_ALL_T)])
    pltpu.sync_copy(dst_hbm.at[pl.ds(s * EDGE_ALL_T, EDGE_ALL_T)],
                    pall.at[pl.ds(PPAD // 2, EDGE_ALL_T)])

    def pos(k, _):
        for u in range(5):
            o = (k * 5 + u) * 16
            sv = pall[pl.ds(o, 16)]
            dv = pall[pl.ds(PPAD // 2 + o, 16)]
            pall[pl.ds(o, 16)] = jnp.where(sv < OUT, dv * OUT + sv, SENT)
            pall[pl.ds(PPAD // 2 + o, 16)] = jnp.where(dv < OUT,
                                                       sv * OUT + dv, SENT)
        return _

    lax.fori_loop(0, EDGE_ALL_T // 80, pos, None)
    for k in range(EDGE_ALL_T // 16, PPAD // 2 // 16):
        pall[pl.ds(k * 16, 16)] = jnp.full((16,), SENT, jnp.int32)
        pall[pl.ds(PPAD // 2 + k * 16, 16)] = jnp.full((16,), SENT, jnp.int32)

    def block(b, _):
        base = (NBLOCK_SC * c + b) * BEL

        for z in range(FTILE // ZCH):
            pltpu.sync_copy(zero_v, mblk.at[pl.ds(s * FTILE + z * ZCH, ZCH)])

        pltpu.sync_copy(zero_v.at[pl.ds(0, 128)],
                        mblk.at[pl.ds(BPAD + s * 128, 128)])

        plsc.subcore_barrier()

        def sb(kb, _):
            def sub(j, _):
                for u in range(8):
                    o = (j * 8 + u) * 16
                    pa = pall[pl.ds(kb * SCHUNK + o, 16)]
                    pr = pa - base
                    ok = (pr >= 0) & (pr < BEL)
                    dummy = BPAD + (pa & 2047)
                    prel[pl.ds(o, 16)] = jnp.where(ok, pr, dummy)
                return _

            lax.fori_loop(0, SCHUNK // 128, sub, None)
            pltpu.sync_copy(ones_v, mblk.at[prel])
            return _

        lax.fori_loop(0, PPAD // SCHUNK, sb, None)
        plsc.subcore_barrier()

        pltpu.sync_copy(mblk.at[pl.ds(s * FTILE, FTILE)],
                        mask_hbm.at[pl.ds(base + s * FTILE, FTILE)])
        return _

    lax.fori_loop(0, NBLOCK_SC, block, None)


@functools.cache
def _mask_kernel():
  return pl.kernel(
    _mask_body,
    out_type=jax.ShapeDtypeStruct((NPAD * OUT,), jnp.float32),
    mesh=_mesh(),
    scratch_types=[
        pltpu.VMEM((PPAD,), jnp.int32),
        pltpu.VMEM((SCHUNK,), jnp.int32),
        pltpu.VMEM((ZCH,), jnp.float32),
        pltpu.VMEM((SCHUNK,), jnp.float32),
        pltpu.VMEM_SHARED((BEL + 2048,), jnp.float32),
    ],
  )


def _mask_call(*args):
    return _mask_kernel()(*args)


def _deg_body(src_hbm, dst_hbm, zeros_hbm, ones_hbm, deg_hbm,
              idx_d, ones_v, acc):
    c = lax.axis_index("c")
    s = lax.axis_index("s")
    pltpu.sync_copy(ones_hbm, ones_v)
    pltpu.sync_copy(zeros_hbm, acc.at[pl.ds(s * ROWS_T, ROWS_T)])
    plsc.subcore_barrier()

    base = c * EDGE_SC + s * EDGE_T

    def batch(k, _):
        off = base + k * DEG_B
        pltpu.sync_copy(dst_hbm.at[pl.ds(off, DEG_B)], idx_d)
        pltpu.sync_copy(ones_v, acc.at[idx_d], add=True)
        return _

    lax.fori_loop(0, DEG_FULL, batch, None)

    plsc.subcore_barrier()
    pltpu.sync_copy(acc.at[pl.ds(s * ROWS_T, ROWS_T)],
                    deg_hbm.at[c, pl.ds(s * ROWS_T, ROWS_T)])


@functools.cache
def _deg_kernel():
  return pl.kernel(
    _deg_body,
    out_type=jax.ShapeDtypeStruct((NSC, NPAD, HID), jnp.float32),
    mesh=_mesh(),
    scratch_types=[
        pltpu.VMEM((DEG_B,), jnp.int32),
        pltpu.VMEM((DEG_B, HID), jnp.float32),
        pltpu.VMEM_SHARED((NPAD, HID), jnp.float32),
    ],
  )


def _deg_call(*args):
    return _deg_kernel()(*args)


def _dot(a, b):
    return jnp.dot(a, b, preferred_element_type=jnp.float32)


def _tc_a_body(x, sfm, tfm, og, op, wl, wr, g1, r1):
    parts = (x[...], sfm[...], tfm[...], og[...], op[...])
    wlv = wl[...]
    wrv = wr[...]
    g = _dot(parts[0], wlv[0:IN])
    r = _dot(parts[0], wrv[0:IN])
    for k in range(1, 5):
        g = g + _dot(parts[k], wlv[k * IN:(k + 1) * IN])
        r = r + _dot(parts[k], wrv[k * IN:(k + 1) * IN])
    g1[...] = g
    r1[...] = r


def _tc_a(x, sfm, tfm, og, op, wl1t, wr1t):
    bs = pl.BlockSpec((BLK, IN), lambda i: (i, 0))
    ws = pl.BlockSpec((5 * IN, HID), lambda i: (0, 0))
    return pl.pallas_call(
        _tc_a_body,
        grid=(NBLK,),
        in_specs=[bs, bs, bs, bs, bs, ws, ws],
        out_specs=[pl.BlockSpec((BLK, HID), lambda i: (i, 0))] * 2,
        out_shape=[jax.ShapeDtypeStruct((N, HID), jnp.float32)] * 2,
    )(x, sfm, tfm, og, op, wl1t, wr1t)


def _deg_stats(degp):
    deg = degp[0, :, 0] + degp[1, :, 0]
    invdeg = 1.0 / jnp.maximum(deg, 1.0)
    degpos = (deg > 0).astype(jnp.float32)
    return invdeg, degpos


def _layernorm(h, g, b):
    m = jnp.mean(h, axis=-1, keepdims=True)
    v = jnp.var(h, axis=-1, keepdims=True)
    return (h - m) / jnp.sqrt(v + 1e-5) * g + b


def _tc_b_body(s1, degp, r1, tfm, bl1, g1n, b1n, wl2a, wr2a, g2, r2, t):
    i = pl.program_id(0)
    invdeg, _ = _deg_stats(degp[...])
    pre = (s1[0] + s1[1]) * invdeg[:, None] + bl1[...] + r1[...]
    h1 = _layernorm(jax.nn.relu(pre), g1n[...], b1n[...])
    g2[...] = _dot(h1, wl2a[...])
    r2[...] = _dot(h1, wr2a[...])
    flag = jnp.any(tfm[...] != 0, axis=1).astype(jnp.float32)
    tp = _dot(flag[None, :], h1)

    @pl.when(i == 0)
    def _init():
        t[...] = jnp.zeros_like(t)

    t[...] += tp


def _tc_b(s1, degp, r1, tfm, bl1, g1n, b1n, wl2at, wr2at):
    bs = pl.BlockSpec((BLK, HID), lambda i: (i, 0))
    ws = pl.BlockSpec((HID, HID), lambda i: (0, 0))
    vs = pl.BlockSpec((1, HID), lambda i: (0, 0))
    return pl.pallas_call(
        _tc_b_body,
        grid=(NBLK,),
        in_specs=[pl.BlockSpec((NSC, BLK, HID), lambda i: (0, i, 0)),
                  pl.BlockSpec((NSC, BLK, HID), lambda i: (0, i, 0)),
                  bs, bs, vs, vs, vs, ws, ws],
        out_specs=[bs, bs, vs],
        out_shape=[jax.ShapeDtypeStruct((N, HID), jnp.float32),
                   jax.ShapeDtypeStruct((N, HID), jnp.float32),
                   jax.ShapeDtypeStruct((1, HID), jnp.float32)],
    )(s1, degp, r1, tfm, bl1, g1n, b1n, wl2at, wr2at)


def _tc_c_body(s2, degp, r2, t, wl2b, wr2b, bl2, g2n, b2n, wl3, wr3, g3, r3):
    invdeg, degpos = _deg_stats(degp[...])
    tv = t[...]
    tl = _dot(tv, wl2b[...])
    tr = _dot(tv, wr2b[...])
    pre = ((s2[0] + s2[1]) * invdeg[:, None] + degpos[:, None] * tl
           + bl2[...] + r2[...] + tr)
    h2 = _layernorm(jax.nn.relu(pre), g2n[...], b2n[...])
    g3[...] = _dot(h2, wl3[...])
    r3[...] = _dot(h2, wr3[...])


def _tc_c(s2, degp, r2, t, wl2bt, wr2bt, bl2, g2n, b2n, wl3t, wr3t):
    bs = pl.BlockSpec((BLK, HID), lambda i: (i, 0))
    ws = pl.BlockSpec((HID, HID), lambda i: (0, 0))
    vs = pl.BlockSpec((1, HID), lambda i: (0, 0))
    return pl.pallas_call(
        _tc_c_body,
        grid=(NBLK,),
        in_specs=[pl.BlockSpec((NSC, BLK, HID), lambda i: (0, i, 0)),
                  pl.BlockSpec((NSC, BLK, HID), lambda i: (0, i, 0)),
                  bs, vs, ws, ws, vs, vs, vs, ws, ws],
        out_specs=[bs, bs],
        out_shape=[jax.ShapeDtypeStruct((N, HID), jnp.float32)] * 2,
    )(s2, degp, r2, t, wl2bt, wr2bt, bl2, g2n, b2n, wl3t, wr3t)


def _tc_d_body(s3, degp, r3, bl3, fcw, fcb, mask, out):
    i = pl.program_id(0)
    invdeg, _ = _deg_stats(degp[...])
    h3 = jax.nn.relu((s3[0] + s3[1]) * invdeg[:, None] + bl3[...] + r3[...])
    logits = _dot(h3, fcw[...]) + fcb[...]
    rid = i * BLK + lax.broadcasted_iota(jnp.int32, (BLK, OUT), 0)
    cid = lax.broadcasted_iota(jnp.int32, (BLK, OUT), 1)
    maskv = jnp.maximum(mask[...], (rid == cid).astype(jnp.float32))
    mx = jnp.max(logits, axis=1, keepdims=True)
    e = jnp.exp(logits - mx) * maskv
    z = jnp.sum(e, axis=1, keepdims=True)
    out[...] = jnp.where(z > 0, e / jnp.where(z > 0, z, 1.0), 0.0)


def _tc_d(s3, degp, r3, bl3, fcwt, fcb, maskm):
    bs = pl.BlockSpec((BLK, HID), lambda i: (i, 0))
    return pl.pallas_call(
        _tc_d_body,
        grid=(NBLK,),
        in_specs=[pl.BlockSpec((NSC, BLK, HID), lambda i: (0, i, 0)),
                  pl.BlockSpec((NSC, BLK, HID), lambda i: (0, i, 0)),
                  bs,
                  pl.BlockSpec((1, HID), lambda i: (0, 0)),
                  pl.BlockSpec((HID, OUT), lambda i: (0, 0)),
                  pl.BlockSpec((1, OUT), lambda i: (0, 0)),
                  pl.BlockSpec((BLK, OUT), lambda i: (i, 0))],
        out_specs=pl.BlockSpec((BLK, OUT), lambda i: (i, 0)),
        out_shape=jax.ShapeDtypeStruct((N, OUT), jnp.float32),
    )(s3, degp, r3, bl3, fcwt, fcb, maskm)


def kernel(x, start_feature_masked, target_feature_masked, other_goals,
           other_pos, edge_index,
           conv1_Wl, conv1_bl, conv1_Wr, conv2_Wl, conv2_bl, conv2_Wr,
           conv3_Wl, conv3_bl, conv3_Wr, fc_W, fc_b,
           ln1_g, ln1_b, ln2_g, ln2_b):
    src = edge_index[0]
    dst = edge_index[1]

    wl1t = conv1_Wl.T
    wr1t = conv1_Wr.T
    wl2at = conv2_Wl[:, :HID].T
    wl2bt = conv2_Wl[:, HID:].T
    wr2at = conv2_Wr[:, :HID].T
    wr2bt = conv2_Wr[:, HID:].T
    wl3t = conv3_Wl.T
    wr3t = conv3_Wr.T
    fcwt = fc_W.T

    bl1 = conv1_bl.reshape(1, HID)
    bl2 = conv2_bl.reshape(1, HID)
    bl3 = conv3_bl.reshape(1, HID)
    fcb = fc_b.reshape(1, OUT)
    g1n = ln1_g.reshape(1, HID)
    b1n = ln1_b.reshape(1, HID)
    g2n = ln2_g.reshape(1, HID)
    b2n = ln2_b.reshape(1, HID)

    zeros_a = jnp.zeros((ROWS_T, HID), jnp.float32)
    zeros_m = jnp.zeros((ZCH,), jnp.float32)
    ones_d = jnp.ones((DEG_B, HID), jnp.float32)
    ones_m = jnp.ones((SCHUNK,), jnp.float32)

    mask_flat = _mask_call(src, dst, zeros_m, ones_m)
    degp = _deg_call(src, dst, zeros_a, ones_d)
    maskm = mask_flat.reshape(NPAD, OUT)

    g1, r1 = _tc_a(x, start_feature_masked, target_feature_masked,
                   other_goals, other_pos, wl1t, wr1t)
    s1 = _seg_call(g1, src, dst, zeros_a)
    g2, r2, t = _tc_b(s1, degp, r1, target_feature_masked, bl1, g1n, b1n,
                      wl2at, wr2at)
    s2 = _seg_call(g2, src, dst, zeros_a)
    g3, r3 = _tc_c(s2, degp, r2, t, wl2bt, wr2bt, bl2, g2n, b2n, wl3t, wr3t)
    s3 = _seg_call(g3, src, dst, zeros_a)
    return _tc_d(s3, degp, r3, bl3, fcwt, fcb, maskm)


# speedup vs baseline: 1.6691x; 1.1661x over previous
import functools

import jax
import jax.numpy as jnp
from jax import lax
from jax.experimental import pallas as pl
from jax.experimental.pallas import tpu as pltpu
from jax.experimental.pallas import tpu_sc as plsc

N = 10000
E = 160000
IN = 128
HID = 128
OUT = 2048

NSC = 2
NT = 16
NPAD = 10240
ROWS_T = NPAD // NT
EDGE_SC = E // NSC
EDGE_T = EDGE_SC // NT
SEG_B = 200
SEG_FULL = EDGE_T // SEG_B

EDGE_ALL_T = E // NT

DEG_B = 200
DEG_FULL = EDGE_T // DEG_B

BROWS = 640
NBLOCK_SC = 8
BEL = BROWS * OUT
BPAD = BEL
ZCH = 16384
SCHUNK = 4096
FTILE = BEL // NT
PPAD = 20480
SENT = 1 << 30

BLK = 400
NBLK = N // BLK

@functools.cache
def _mesh():
    return plsc.VectorSubcoreMesh(core_axis_name="c", subcore_axis_name="s")


def _seg_body(g_hbm, src_hbm, dst_hbm, zeros_hbm, out_hbm,
              idx_s, idx_d, rows, acc, sem):
    c = lax.axis_index("c")
    s = lax.axis_index("s")
    pltpu.sync_copy(zeros_hbm, acc.at[pl.ds(s * ROWS_T, ROWS_T)])
    plsc.subcore_barrier()

    base = c * EDGE_SC + s * EDGE_T

    def batch(k, _):
        off = base + k * SEG_B
        pltpu.sync_copy(src_hbm.at[pl.ds(off, SEG_B)], idx_s)
        pltpu.sync_copy(dst_hbm.at[pl.ds(off, SEG_B)], idx_d)
        pltpu.async_copy(g_hbm.at[idx_s], rows, sem).wait()
        pltpu.sync_copy(rows, acc.at[idx_d], add=True)
        return _

    lax.fori_loop(0, SEG_FULL, batch, None)

    plsc.subcore_barrier()
    pltpu.sync_copy(acc.at[pl.ds(s * ROWS_T, ROWS_T)],
                    out_hbm.at[c, pl.ds(s * ROWS_T, ROWS_T)])


@functools.cache
def _seg_kernel():
  return pl.kernel(
    _seg_body,
    out_type=jax.ShapeDtypeStruct((NSC, NPAD, HID), jnp.float32),
    mesh=_mesh(),
    scratch_types=[
        pltpu.VMEM((SEG_B,), jnp.int32),
        pltpu.VMEM((SEG_B,), jnp.int32),
        pltpu.VMEM((SEG_B, HID), jnp.float32),
        pltpu.VMEM_SHARED((NPAD, HID), jnp.float32),
        pltpu.SemaphoreType.DMA,
    ],
  )


def _seg_call(*args):
    return _seg_kernel()(*args)


def _mask_body(src_hbm, dst_hbm, zeros_hbm, ones_hbm, mask_hbm,
               pall, prel, zero_v, ones_v, mblk):
    c = lax.axis_index("c")
    s = lax.axis_index("s")

    pltpu.sync_copy(zeros_hbm, zero_v)
    pltpu.sync_copy(ones_hbm, ones_v)
    pltpu.sync_copy(src_hbm.at[pl.ds(s * EDGE_ALL_T, EDGE_ALL_T)],
                    pall.at[pl.ds(0, EDGE_ALL_T)])
    pltpu.sync_copy(dst_hbm.at[pl.ds(s * EDGE_ALL_T, EDGE_ALL_T)],
                    pall.at[pl.ds(PPAD // 2, EDGE_ALL_T)])

    def pos(k, _):
        for u in range(5):
            o = (k * 5 + u) * 16
            sv = pall[pl.ds(o, 16)]
            dv = pall[pl.ds(PPAD // 2 + o, 16)]
            pall[pl.ds(o, 16)] = jnp.where(sv < OUT, dv * OUT + sv, SENT)
            pall[pl.ds(PPAD // 2 + o, 16)] = jnp.where(dv < OUT,
                                                       sv * OUT + dv, SENT)
        return _

    lax.fori_loop(0, EDGE_ALL_T // 80, pos, None)
    for k in range(EDGE_ALL_T // 16, PPAD // 2 // 16):
        pall[pl.ds(k * 16, 16)] = jnp.full((16,), SENT, jnp.int32)
        pall[pl.ds(PPAD // 2 + k * 16, 16)] = jnp.full((16,), SENT, jnp.int32)

    def block(b, _):
        base = (NBLOCK_SC * c + b) * BEL

        for z in range(FTILE // ZCH):
            pltpu.sync_copy(zero_v, mblk.at[pl.ds(s * FTILE + z * ZCH, ZCH)])

        pltpu.sync_copy(zero_v.at[pl.ds(0, 128)],
                        mblk.at[pl.ds(BPAD + s * 128, 128)])

        plsc.subcore_barrier()

        def sb(kb, _):
            def sub(j, _):
                for u in range(8):
                    o = (j * 8 + u) * 16
                    pa = pall[pl.ds(kb * SCHUNK + o, 16)]
                    pr = pa - base
                    ok = (pr >= 0) & (pr < BEL)
                    dummy = BPAD + (pa & 2047)
                    prel[pl.ds(o, 16)] = jnp.where(ok, pr, dummy)
                return _

            lax.fori_loop(0, SCHUNK // 128, sub, None)
            pltpu.sync_copy(ones_v, mblk.at[prel])
            return _

        lax.fori_loop(0, PPAD // SCHUNK, sb, None)
        plsc.subcore_barrier()

        pltpu.sync_copy(mblk.at[pl.ds(s * FTILE, FTILE)],
                        mask_hbm.at[pl.ds(base + s * FTILE, FTILE)])
        return _

    lax.fori_loop(0, NBLOCK_SC, block, None)


@functools.cache
def _mask_kernel():
  return pl.kernel(
    _mask_body,
    out_type=jax.ShapeDtypeStruct((NPAD * OUT,), jnp.float32),
    mesh=_mesh(),
    scratch_types=[
        pltpu.VMEM((PPAD,), jnp.int32),
        pltpu.VMEM((SCHUNK,), jnp.int32),
        pltpu.VMEM((ZCH,), jnp.float32),
        pltpu.VMEM((SCHUNK,), jnp.float32),
        pltpu.VMEM_SHARED((BEL + 2048,), jnp.float32),
    ],
  )


def _mask_call(*args):
    return _mask_kernel()(*args)


def _deg_body(src_hbm, dst_hbm, zeros_hbm, ones_hbm, deg_hbm,
              idx_d, ones_v, acc):
    c = lax.axis_index("c")
    s = lax.axis_index("s")
    pltpu.sync_copy(ones_hbm, ones_v)
    pltpu.sync_copy(zeros_hbm, acc.at[pl.ds(s * ROWS_T, ROWS_T)])
    plsc.subcore_barrier()

    base = c * EDGE_SC + s * EDGE_T

    def batch(k, _):
        off = base + k * DEG_B
        pltpu.sync_copy(dst_hbm.at[pl.ds(off, DEG_B)], idx_d)
        pltpu.sync_copy(ones_v, acc.at[idx_d], add=True)
        return _

    lax.fori_loop(0, DEG_FULL, batch, None)

    plsc.subcore_barrier()
    pltpu.sync_copy(acc.at[pl.ds(s * ROWS_T, ROWS_T)],
                    deg_hbm.at[c, pl.ds(s * ROWS_T, ROWS_T)])


@functools.cache
def _deg_kernel():
  return pl.kernel(
    _deg_body,
    out_type=jax.ShapeDtypeStruct((NSC, NPAD, HID), jnp.float32),
    mesh=_mesh(),
    scratch_types=[
        pltpu.VMEM((DEG_B,), jnp.int32),
        pltpu.VMEM((DEG_B, HID), jnp.float32),
        pltpu.VMEM_SHARED((NPAD, HID), jnp.float32),
    ],
  )


def _deg_call(*args):
    return _deg_kernel()(*args)


def _dot(a, b):
    return jnp.dot(a, b, preferred_element_type=jnp.float32)


def _tc_a_body(x, sfm, tfm, og, op, wl, wr, g1, r1):
    parts = (x[...], sfm[...], tfm[...], og[...], op[...])
    wlv = wl[...]
    wrv = wr[...]
    g = _dot(parts[0], wlv[0:IN])
    r = _dot(parts[0], wrv[0:IN])
    for k in range(1, 5):
        g = g + _dot(parts[k], wlv[k * IN:(k + 1) * IN])
        r = r + _dot(parts[k], wrv[k * IN:(k + 1) * IN])
    g1[...] = g
    r1[...] = r


def _tc_a(x, sfm, tfm, og, op, wl1t, wr1t):
    bs = pl.BlockSpec((BLK, IN), lambda i: (i, 0))
    ws = pl.BlockSpec((5 * IN, HID), lambda i: (0, 0))
    return pl.pallas_call(
        _tc_a_body,
        grid=(NBLK,),
        in_specs=[bs, bs, bs, bs, bs, ws, ws],
        out_specs=[pl.BlockSpec((BLK, HID), lambda i: (i, 0))] * 2,
        out_shape=[jax.ShapeDtypeStruct((N, HID), jnp.float32)] * 2,
    )(x, sfm, tfm, og, op, wl1t, wr1t)


def _deg_stats(degp):
    deg = degp[0, :, 0] + degp[1, :, 0]
    invdeg = 1.0 / jnp.maximum(deg, 1.0)
    degpos = (deg > 0).astype(jnp.float32)
    return invdeg, degpos


def _layernorm(h, g, b):
    m = jnp.mean(h, axis=-1, keepdims=True)
    v = jnp.var(h, axis=-1, keepdims=True)
    return (h - m) / jnp.sqrt(v + 1e-5) * g + b


def _tc_b_body(s1, degp, r1, tfm, bl1, g1n, b1n, wl2a, wr2a, g2, r2, t):
    i = pl.program_id(0)
    invdeg, _ = _deg_stats(degp[...])
    pre = (s1[0] + s1[1]) * invdeg[:, None] + bl1[...] + r1[...]
    h1 = _layernorm(jax.nn.relu(pre), g1n[...], b1n[...])
    g2[...] = _dot(h1, wl2a[...])
    r2[...] = _dot(h1, wr2a[...])
    flag = jnp.any(tfm[...] != 0, axis=1).astype(jnp.float32)
    tp = _dot(flag[None, :], h1)

    @pl.when(i == 0)
    def _init():
        t[...] = jnp.zeros_like(t)

    t[...] += tp


def _tc_b(s1, degp, r1, tfm, bl1, g1n, b1n, wl2at, wr2at):
    bs = pl.BlockSpec((BLK, HID), lambda i: (i, 0))
    ws = pl.BlockSpec((HID, HID), lambda i: (0, 0))
    vs = pl.BlockSpec((1, HID), lambda i: (0, 0))
    return pl.pallas_call(
        _tc_b_body,
        grid=(NBLK,),
        in_specs=[pl.BlockSpec((NSC, BLK, HID), lambda i: (0, i, 0)),
                  pl.BlockSpec((NSC, BLK, HID), lambda i: (0, i, 0)),
                  bs, bs, vs, vs, vs, ws, ws],
        out_specs=[bs, bs, vs],
        out_shape=[jax.ShapeDtypeStruct((N, HID), jnp.float32),
                   jax.ShapeDtypeStruct((N, HID), jnp.float32),
                   jax.ShapeDtypeStruct((1, HID), jnp.float32)],
    )(s1, degp, r1, tfm, bl1, g1n, b1n, wl2at, wr2at)


def _tc_c_body(s2, degp, r2, t, wl2b, wr2b, bl2, g2n, b2n, wl3, wr3, g3, r3):
    invdeg, degpos = _deg_stats(degp[...])
    tv = t[...]
    tl = _dot(tv, wl2b[...])
    tr = _dot(tv, wr2b[...])
    pre = ((s2[0] + s2[1]) * invdeg[:, None] + degpos[:, None] * tl
           + bl2[...] + r2[...] + tr)
    h2 = _layernorm(jax.nn.relu(pre), g2n[...], b2n[...])
    g3[...] = _dot(h2, wl3[...])
    r3[...] = _dot(h2, wr3[...])


def _tc_c(s2, degp, r2, t, wl2bt, wr2bt, bl2, g2n, b2n, wl3t, wr3t):
    bs = pl.BlockSpec((BLK, HID), lambda i: (i, 0))
    ws = pl.BlockSpec((HID, HID), lambda i: (0, 0))
    vs = pl.BlockSpec((1, HID), lambda i: (0, 0))
    return pl.pallas_call(
        _tc_c_body,
        grid=(NBLK,),
        in_specs=[pl.BlockSpec((NSC, BLK, HID), lambda i: (0, i, 0)),
                  pl.BlockSpec((NSC, BLK, HID), lambda i: (0, i, 0)),
                  bs, vs, ws, ws, vs, vs, vs, ws, ws],
        out_specs=[bs, bs],
        out_shape=[jax.ShapeDtypeStruct((N, HID), jnp.float32)] * 2,
    )(s2, degp, r2, t, wl2bt, wr2bt, bl2, g2n, b2n, wl3t, wr3t)


def _tc_d_body(s3, degp, r3, bl3, fcw, fcb, mask, out):
    i = pl.program_id(0)
    invdeg, _ = _deg_stats(degp[...])
    h3 = jax.nn.relu((s3[0] + s3[1]) * invdeg[:, None] + bl3[...] + r3[...])
    logits = _dot(h3, fcw[...]) + fcb[...]
    rid = i * BLK + lax.broadcasted_iota(jnp.int32, (BLK, OUT), 0)
    cid = lax.broadcasted_iota(jnp.int32, (BLK, OUT), 1)
    maskv = jnp.maximum(mask[...], (rid == cid).astype(jnp.float32))
    mx = jnp.max(logits, axis=1, keepdims=True)
    e = jnp.exp(logits - mx) * maskv
    z = jnp.sum(e, axis=1, keepdims=True)
    out[...] = jnp.where(z > 0, e / jnp.where(z > 0, z, 1.0), 0.0)


def _tc_d(s3, degp, r3, bl3, fcwt, fcb, maskm):
    bs = pl.BlockSpec((BLK, HID), lambda i: (i, 0))
    return pl.pallas_call(
        _tc_d_body,
        grid=(NBLK,),
        in_specs=[pl.BlockSpec((NSC, BLK, HID), lambda i: (0, i, 0)),
                  pl.BlockSpec((NSC, BLK, HID), lambda i: (0, i, 0)),
                  bs,
                  pl.BlockSpec((1, HID), lambda i: (0, 0)),
                  pl.BlockSpec((HID, OUT), lambda i: (0, 0)),
                  pl.BlockSpec((1, OUT), lambda i: (0, 0)),
                  pl.BlockSpec((BLK, OUT), lambda i: (i, 0))],
        out_specs=pl.BlockSpec((BLK, OUT), lambda i: (i, 0)),
        out_shape=jax.ShapeDtypeStruct((N, OUT), jnp.float32),
    )(s3, degp, r3, bl3, fcwt, fcb, maskm)


def kernel(x, start_feature_masked, target_feature_masked, other_goals,
           other_pos, edge_index,
           conv1_Wl, conv1_bl, conv1_Wr, conv2_Wl, conv2_bl, conv2_Wr,
           conv3_Wl, conv3_bl, conv3_Wr, fc_W, fc_b,
           ln1_g, ln1_b, ln2_g, ln2_b):
    src = edge_index[0]
    dst = edge_index[1]

    wl1t = conv1_Wl.T
    wr1t = conv1_Wr.T
    wl2at = conv2_Wl[:, :HID].T
    wl2bt = conv2_Wl[:, HID:].T
    wr2at = conv2_Wr[:, :HID].T
    wr2bt = conv2_Wr[:, HID:].T
    wl3t = conv3_Wl.T
    wr3t = conv3_Wr.T
    fcwt = fc_W.T

    bl1 = conv1_bl.reshape(1, HID)
    bl2 = conv2_bl.reshape(1, HID)
    bl3 = conv3_bl.reshape(1, HID)
    fcb = fc_b.reshape(1, OUT)
    g1n = ln1_g.reshape(1, HID)
    b1n = ln1_b.reshape(1, HID)
    g2n = ln2_g.reshape(1, HID)
    b2n = ln2_b.reshape(1, HID)

    zeros_a = jnp.zeros((ROWS_T, HID), jnp.float32)
    zeros_m = jnp.zeros((ZCH,), jnp.float32)
    ones_d = jnp.ones((DEG_B, HID), jnp.float32)
    ones_m = jnp.ones((SCHUNK,), jnp.float32)

    mask_flat = _mask_call(src, dst, zeros_m, ones_m)
    degp = _deg_call(src, dst, zeros_a, ones_d)
    maskm = mask_flat.reshape(NPAD, OUT)

    g1, r1 = _tc_a(x, start_feature_masked, target_feature_masked,
                   other_goals, other_pos, wl1t, wr1t)
    s1 = _seg_call(g1, src, dst, zeros_a)
    g2, r2, t = _tc_b(s1, degp, r1, target_feature_masked, bl1, g1n, b1n,
                      wl2at, wr2at)
    s2 = _seg_call(g2, src, dst, zeros_a)
    g3, r3 = _tc_c(s2, degp, r2, t, wl2bt, wr2bt, bl2, g2n, b2n, wl3t, wr3t)
    s3 = _seg_call(g3, src, dst, zeros_a)
    return _tc_d(s3, degp, r3, bl3, fcwt, fcb, maskm)


# speedup vs baseline: 1.7311x; 1.0372x over previous
import functools

import jax
import jax.numpy as jnp
from jax import lax
from jax.experimental import pallas as pl
from jax.experimental.pallas import tpu as pltpu
from jax.experimental.pallas import tpu_sc as plsc

N = 10000
E = 160000
IN = 128
HID = 128
OUT = 2048

NSC = 2
NT = 16
NPAD = 10240
ROWS_T = NPAD // NT
EDGE_SC = E // NSC
EDGE_T = EDGE_SC // NT
SEG_B = 200
SEG_FULL = EDGE_T // SEG_B

EDGE_ALL_T = E // NT

DEG_B = 200
DEG_FULL = EDGE_T // DEG_B

BROWS = 640
NBLOCK_SC = 8
BEL = BROWS * OUT
BPAD = BEL
ZCH = 16384
SCHUNK = 4096
FTILE = BEL // NT
PPAD = 20480
SENT = 1 << 30

BLK = 400
NBLK = N // BLK

@functools.cache
def _mesh():
    return plsc.VectorSubcoreMesh(core_axis_name="c", subcore_axis_name="s")


def _seg_body(g_hbm, src_hbm, dst_hbm, zeros_hbm, out_hbm,
              idx_s0, idx_d0, idx_s1, idx_d1, rows, acc, semi, semg):
    c = lax.axis_index("c")
    s = lax.axis_index("s")
    pltpu.sync_copy(zeros_hbm, acc.at[pl.ds(s * ROWS_T, ROWS_T)])
    plsc.subcore_barrier()

    base = c * EDGE_SC + s * EDGE_T
    last = base + (SEG_FULL - 1) * SEG_B

    def _di(sem, dst):
        pltpu.make_async_copy(src_hbm.at[pl.ds(0, SEG_B)], dst, sem).wait()

    pltpu.sync_copy(src_hbm.at[pl.ds(base, SEG_B)], idx_s0)
    pltpu.sync_copy(dst_hbm.at[pl.ds(base, SEG_B)], idx_d0)

    def one(off_next, idx_sc, idx_dc, idx_sn, idx_dn):
        pltpu.async_copy(src_hbm.at[pl.ds(off_next, SEG_B)], idx_sn, semi)
        pltpu.async_copy(dst_hbm.at[pl.ds(off_next, SEG_B)], idx_dn, semi)
        pltpu.async_copy(g_hbm.at[idx_sc], rows, semg).wait()
        pltpu.sync_copy(rows, acc.at[idx_dc], add=True)
        _di(semi, idx_sn)
        _di(semi, idx_dn)

    def pair(k2, _):
        off1 = base + (2 * k2 + 1) * SEG_B
        off2 = jnp.minimum(base + (2 * k2 + 2) * SEG_B, last)
        one(off1, idx_s0, idx_d0, idx_s1, idx_d1)
        one(off2, idx_s1, idx_d1, idx_s0, idx_d0)
        return _

    lax.fori_loop(0, (SEG_FULL - 1) // 2, pair, None)
    pltpu.async_copy(g_hbm.at[idx_s0], rows, semg).wait()
    pltpu.sync_copy(rows, acc.at[idx_d0], add=True)

    plsc.subcore_barrier()
    pltpu.sync_copy(acc.at[pl.ds(s * ROWS_T, ROWS_T)],
                    out_hbm.at[c, pl.ds(s * ROWS_T, ROWS_T)])


@functools.cache
def _seg_kernel():
  return pl.kernel(
    _seg_body,
    out_type=jax.ShapeDtypeStruct((NSC, NPAD, HID), jnp.float32),
    mesh=_mesh(),
    scratch_types=[
        pltpu.VMEM((SEG_B,), jnp.int32),
        pltpu.VMEM((SEG_B,), jnp.int32),
        pltpu.VMEM((SEG_B,), jnp.int32),
        pltpu.VMEM((SEG_B,), jnp.int32),
        pltpu.VMEM((SEG_B, HID), jnp.float32),
        pltpu.VMEM_SHARED((NPAD, HID), jnp.float32),
        pltpu.SemaphoreType.DMA,
        pltpu.SemaphoreType.DMA,
    ],
  )


def _seg_call(*args):
    return _seg_kernel()(*args)


def _mask_body(src_hbm, dst_hbm, zeros_hbm, ones_hbm, mask_hbm,
               pall, prel, zero_v, ones_v, mblk):
    c = lax.axis_index("c")
    s = lax.axis_index("s")

    pltpu.sync_copy(zeros_hbm, zero_v)
    pltpu.sync_copy(ones_hbm, ones_v)
    pltpu.sync_copy(src_hbm.at[pl.ds(s * EDGE_ALL_T, EDGE_ALL_T)],
                    pall.at[pl.ds(0, EDGE_ALL_T)])
    pltpu.sync_copy(dst_hbm.at[pl.ds(s * EDGE_ALL_T, EDGE_ALL_T)],
                    pall.at[pl.ds(PPAD // 2, EDGE_ALL_T)])

    def pos(k, _):
        for u in range(5):
            o = (k * 5 + u) * 16
            sv = pall[pl.ds(o, 16)]
            dv = pall[pl.ds(PPAD // 2 + o, 16)]
            pall[pl.ds(o, 16)] = jnp.where(sv < OUT, dv * OUT + sv, SENT)
            pall[pl.ds(PPAD // 2 + o, 16)] = jnp.where(dv < OUT,
                                                       sv * OUT + dv, SENT)
        return _

    lax.fori_loop(0, EDGE_ALL_T // 80, pos, None)
    for k in range(EDGE_ALL_T // 16, PPAD // 2 // 16):
        pall[pl.ds(k * 16, 16)] = jnp.full((16,), SENT, jnp.int32)
        pall[pl.ds(PPAD // 2 + k * 16, 16)] = jnp.full((16,), SENT, jnp.int32)

    def block(b, _):
        base = (NBLOCK_SC * c + b) * BEL

        for z in range(FTILE // ZCH):
            pltpu.sync_copy(zero_v, mblk.at[pl.ds(s * FTILE + z * ZCH, ZCH)])

        pltpu.sync_copy(zero_v.at[pl.ds(0, 128)],
                        mblk.at[pl.ds(BPAD + s * 128, 128)])

        plsc.subcore_barrier()

        def sb(kb, _):
            def sub(j, _):
                for u in range(8):
                    o = (j * 8 + u) * 16
                    pa = pall[pl.ds(kb * SCHUNK + o, 16)]
                    pr = pa - base
                    ok = (pr >= 0) & (pr < BEL)
                    dummy = BPAD + (pa & 2047)
                    prel[pl.ds(o, 16)] = jnp.where(ok, pr, dummy)
                return _

            lax.fori_loop(0, SCHUNK // 128, sub, None)
            pltpu.sync_copy(ones_v, mblk.at[prel])
            return _

        lax.fori_loop(0, PPAD // SCHUNK, sb, None)
        plsc.subcore_barrier()

        pltpu.sync_copy(mblk.at[pl.ds(s * FTILE, FTILE)],
                        mask_hbm.at[pl.ds(base + s * FTILE, FTILE)])
        return _

    lax.fori_loop(0, NBLOCK_SC, block, None)


@functools.cache
def _mask_kernel():
  return pl.kernel(
    _mask_body,
    out_type=jax.ShapeDtypeStruct((NPAD * OUT,), jnp.float32),
    mesh=_mesh(),
    scratch_types=[
        pltpu.VMEM((PPAD,), jnp.int32),
        pltpu.VMEM((SCHUNK,), jnp.int32),
        pltpu.VMEM((ZCH,), jnp.float32),
        pltpu.VMEM((SCHUNK,), jnp.float32),
        pltpu.VMEM_SHARED((BEL + 2048,), jnp.float32),
    ],
  )


def _mask_call(*args):
    return _mask_kernel()(*args)


def _deg_body(src_hbm, dst_hbm, zeros_hbm, ones_hbm, deg_hbm,
              idx_d, ones_v, acc):
    c = lax.axis_index("c")
    s = lax.axis_index("s")
    pltpu.sync_copy(ones_hbm, ones_v)
    pltpu.sync_copy(zeros_hbm, acc.at[pl.ds(s * ROWS_T, ROWS_T)])
    plsc.subcore_barrier()

    base = c * EDGE_SC + s * EDGE_T

    def batch(k, _):
        off = base + k * DEG_B
        pltpu.sync_copy(dst_hbm.at[pl.ds(off, DEG_B)], idx_d)
        pltpu.sync_copy(ones_v, acc.at[idx_d], add=True)
        return _

    lax.fori_loop(0, DEG_FULL, batch, None)

    plsc.subcore_barrier()
    pltpu.sync_copy(acc.at[pl.ds(s * ROWS_T, ROWS_T)],
                    deg_hbm.at[c, pl.ds(s * ROWS_T, ROWS_T)])


@functools.cache
def _deg_kernel():
  return pl.kernel(
    _deg_body,
    out_type=jax.ShapeDtypeStruct((NSC, NPAD, HID), jnp.float32),
    mesh=_mesh(),
    scratch_types=[
        pltpu.VMEM((DEG_B,), jnp.int32),
        pltpu.VMEM((DEG_B, HID), jnp.float32),
        pltpu.VMEM_SHARED((NPAD, HID), jnp.float32),
    ],
  )


def _deg_call(*args):
    return _deg_kernel()(*args)


def _dot(a, b):
    return jnp.dot(a, b, preferred_element_type=jnp.float32)


def _tc_a_body(x, sfm, tfm, og, op, wl, wr, g1, r1):
    parts = (x[...], sfm[...], tfm[...], og[...], op[...])
    wlv = wl[...]
    wrv = wr[...]
    g = _dot(parts[0], wlv[0:IN])
    r = _dot(parts[0], wrv[0:IN])
    for k in range(1, 5):
        g = g + _dot(parts[k], wlv[k * IN:(k + 1) * IN])
        r = r + _dot(parts[k], wrv[k * IN:(k + 1) * IN])
    g1[...] = g
    r1[...] = r


def _tc_a(x, sfm, tfm, og, op, wl1t, wr1t):
    bs = pl.BlockSpec((BLK, IN), lambda i: (i, 0))
    ws = pl.BlockSpec((5 * IN, HID), lambda i: (0, 0))
    return pl.pallas_call(
        _tc_a_body,
        grid=(NBLK,),
        in_specs=[bs, bs, bs, bs, bs, ws, ws],
        out_specs=[pl.BlockSpec((BLK, HID), lambda i: (i, 0))] * 2,
        out_shape=[jax.ShapeDtypeStruct((N, HID), jnp.float32)] * 2,
    )(x, sfm, tfm, og, op, wl1t, wr1t)


def _deg_stats(degp):
    deg = degp[0, :, 0] + degp[1, :, 0]
    invdeg = 1.0 / jnp.maximum(deg, 1.0)
    degpos = (deg > 0).astype(jnp.float32)
    return invdeg, degpos


def _layernorm(h, g, b):
    m = jnp.mean(h, axis=-1, keepdims=True)
    v = jnp.var(h, axis=-1, keepdims=True)
    return (h - m) / jnp.sqrt(v + 1e-5) * g + b


def _tc_b_body(s1, degp, r1, tfm, bl1, g1n, b1n, wl2a, wr2a, g2, r2, t):
    i = pl.program_id(0)
    invdeg, _ = _deg_stats(degp[...])
    pre = (s1[0] + s1[1]) * invdeg[:, None] + bl1[...] + r1[...]
    h1 = _layernorm(jax.nn.relu(pre), g1n[...], b1n[...])
    g2[...] = _dot(h1, wl2a[...])
    r2[...] = _dot(h1, wr2a[...])
    flag = jnp.any(tfm[...] != 0, axis=1).astype(jnp.float32)
    tp = _dot(flag[None, :], h1)

    @pl.when(i == 0)
    def _init():
        t[...] = jnp.zeros_like(t)

    t[...] += tp


def _tc_b(s1, degp, r1, tfm, bl1, g1n, b1n, wl2at, wr2at):
    bs = pl.BlockSpec((BLK, HID), lambda i: (i, 0))
    ws = pl.BlockSpec((HID, HID), lambda i: (0, 0))
    vs = pl.BlockSpec((1, HID), lambda i: (0, 0))
    return pl.pallas_call(
        _tc_b_body,
        grid=(NBLK,),
        in_specs=[pl.BlockSpec((NSC, BLK, HID), lambda i: (0, i, 0)),
                  pl.BlockSpec((NSC, BLK, HID), lambda i: (0, i, 0)),
                  bs, bs, vs, vs, vs, ws, ws],
        out_specs=[bs, bs, vs],
        out_shape=[jax.ShapeDtypeStruct((N, HID), jnp.float32),
                   jax.ShapeDtypeStruct((N, HID), jnp.float32),
                   jax.ShapeDtypeStruct((1, HID), jnp.float32)],
    )(s1, degp, r1, tfm, bl1, g1n, b1n, wl2at, wr2at)


def _tc_c_body(s2, degp, r2, t, wl2b, wr2b, bl2, g2n, b2n, wl3, wr3, g3, r3):
    invdeg, degpos = _deg_stats(degp[...])
    tv = t[...]
    tl = _dot(tv, wl2b[...])
    tr = _dot(tv, wr2b[...])
    pre = ((s2[0] + s2[1]) * invdeg[:, None] + degpos[:, None] * tl
           + bl2[...] + r2[...] + tr)
    h2 = _layernorm(jax.nn.relu(pre), g2n[...], b2n[...])
    g3[...] = _dot(h2, wl3[...])
    r3[...] = _dot(h2, wr3[...])


def _tc_c(s2, degp, r2, t, wl2bt, wr2bt, bl2, g2n, b2n, wl3t, wr3t):
    bs = pl.BlockSpec((BLK, HID), lambda i: (i, 0))
    ws = pl.BlockSpec((HID, HID), lambda i: (0, 0))
    vs = pl.BlockSpec((1, HID), lambda i: (0, 0))
    return pl.pallas_call(
        _tc_c_body,
        grid=(NBLK,),
        in_specs=[pl.BlockSpec((NSC, BLK, HID), lambda i: (0, i, 0)),
                  pl.BlockSpec((NSC, BLK, HID), lambda i: (0, i, 0)),
                  bs, vs, ws, ws, vs, vs, vs, ws, ws],
        out_specs=[bs, bs],
        out_shape=[jax.ShapeDtypeStruct((N, HID), jnp.float32)] * 2,
    )(s2, degp, r2, t, wl2bt, wr2bt, bl2, g2n, b2n, wl3t, wr3t)


def _tc_d_body(s3, degp, r3, bl3, fcw, fcb, mask, out):
    i = pl.program_id(0)
    invdeg, _ = _deg_stats(degp[...])
    h3 = jax.nn.relu((s3[0] + s3[1]) * invdeg[:, None] + bl3[...] + r3[...])
    logits = _dot(h3, fcw[...]) + fcb[...]
    rid = i * BLK + lax.broadcasted_iota(jnp.int32, (BLK, OUT), 0)
    cid = lax.broadcasted_iota(jnp.int32, (BLK, OUT), 1)
    maskv = jnp.maximum(mask[...], (rid == cid).astype(jnp.float32))
    mx = jnp.max(logits, axis=1, keepdims=True)
    e = jnp.exp(logits - mx) * maskv
    z = jnp.sum(e, axis=1, keepdims=True)
    out[...] = jnp.where(z > 0, e / jnp.where(z > 0, z, 1.0), 0.0)


def _tc_d(s3, degp, r3, bl3, fcwt, fcb, maskm):
    bs = pl.BlockSpec((BLK, HID), lambda i: (i, 0))
    return pl.pallas_call(
        _tc_d_body,
        grid=(NBLK,),
        in_specs=[pl.BlockSpec((NSC, BLK, HID), lambda i: (0, i, 0)),
                  pl.BlockSpec((NSC, BLK, HID), lambda i: (0, i, 0)),
                  bs,
                  pl.BlockSpec((1, HID), lambda i: (0, 0)),
                  pl.BlockSpec((HID, OUT), lambda i: (0, 0)),
                  pl.BlockSpec((1, OUT), lambda i: (0, 0)),
                  pl.BlockSpec((BLK, OUT), lambda i: (i, 0))],
        out_specs=pl.BlockSpec((BLK, OUT), lambda i: (i, 0)),
        out_shape=jax.ShapeDtypeStruct((N, OUT), jnp.float32),
    )(s3, degp, r3, bl3, fcwt, fcb, maskm)


def kernel(x, start_feature_masked, target_feature_masked, other_goals,
           other_pos, edge_index,
           conv1_Wl, conv1_bl, conv1_Wr, conv2_Wl, conv2_bl, conv2_Wr,
           conv3_Wl, conv3_bl, conv3_Wr, fc_W, fc_b,
           ln1_g, ln1_b, ln2_g, ln2_b):
    src = edge_index[0]
    dst = edge_index[1]

    wl1t = conv1_Wl.T
    wr1t = conv1_Wr.T
    wl2at = conv2_Wl[:, :HID].T
    wl2bt = conv2_Wl[:, HID:].T
    wr2at = conv2_Wr[:, :HID].T
    wr2bt = conv2_Wr[:, HID:].T
    wl3t = conv3_Wl.T
    wr3t = conv3_Wr.T
    fcwt = fc_W.T

    bl1 = conv1_bl.reshape(1, HID)
    bl2 = conv2_bl.reshape(1, HID)
    bl3 = conv3_bl.reshape(1, HID)
    fcb = fc_b.reshape(1, OUT)
    g1n = ln1_g.reshape(1, HID)
    b1n = ln1_b.reshape(1, HID)
    g2n = ln2_g.reshape(1, HID)
    b2n = ln2_b.reshape(1, HID)

    zeros_a = jnp.zeros((ROWS_T, HID), jnp.float32)
    zeros_m = jnp.zeros((ZCH,), jnp.float32)
    ones_d = jnp.ones((DEG_B, HID), jnp.float32)
    ones_m = jnp.ones((SCHUNK,), jnp.float32)

    mask_flat = _mask_call(src, dst, zeros_m, ones_m)
    degp = _deg_call(src, dst, zeros_a, ones_d)
    maskm = mask_flat.reshape(NPAD, OUT)

    g1, r1 = _tc_a(x, start_feature_masked, target_feature_masked,
                   other_goals, other_pos, wl1t, wr1t)
    s1 = _seg_call(g1, src, dst, zeros_a)
    g2, r2, t = _tc_b(s1, degp, r1, target_feature_masked, bl1, g1n, b1n,
                      wl2at, wr2at)
    s2 = _seg_call(g2, src, dst, zeros_a)
    g3, r3 = _tc_c(s2, degp, r2, t, wl2bt, wr2bt, bl2, g2n, b2n, wl3t, wr3t)
    s3 = _seg_call(g3, src, dst, zeros_a)
    return _tc_d(s3, degp, r3, bl3, fcwt, fcb, maskm)


# speedup vs baseline: 1.7394x; 1.0048x over previous
import functools

import jax
import jax.numpy as jnp
from jax import lax
from jax.experimental import pallas as pl
from jax.experimental.pallas import tpu as pltpu
from jax.experimental.pallas import tpu_sc as plsc

N = 10000
E = 160000
IN = 128
HID = 128
OUT = 2048

NSC = 2
NT = 16
NPAD = 10240
ROWS_T = NPAD // NT
EDGE_SC = E // NSC
EDGE_T = EDGE_SC // NT
SEG_B = 200
SEG_FULL = EDGE_T // SEG_B

EDGE_ALL_T = E // NT

DEG_B = 200
DEG_FULL = EDGE_T // DEG_B

BROWS = 640
NBLOCK_SC = 8
BEL = BROWS * OUT
BPAD = BEL
ZCH = 8192
SCHUNK = 4096
FTILE = BEL // NT
PPAD = 20480
SENT = 1 << 30

BLK = 400
NBLK = N // BLK

@functools.cache
def _mesh():
    return plsc.VectorSubcoreMesh(core_axis_name="c", subcore_axis_name="s")


def _seg_body(g_hbm, src_hbm, dst_hbm, zeros_hbm, out_hbm,
              idx_s0, idx_d0, idx_s1, idx_d1, rows, acc, semi, semg):
    c = lax.axis_index("c")
    s = lax.axis_index("s")
    pltpu.sync_copy(zeros_hbm, acc.at[pl.ds(s * ROWS_T, ROWS_T)])
    plsc.subcore_barrier()

    base = c * EDGE_SC + s * EDGE_T
    last = base + (SEG_FULL - 1) * SEG_B

    def _di(sem, dst):
        pltpu.make_async_copy(src_hbm.at[pl.ds(0, SEG_B)], dst, sem).wait()

    pltpu.sync_copy(src_hbm.at[pl.ds(base, SEG_B)], idx_s0)
    pltpu.sync_copy(dst_hbm.at[pl.ds(base, SEG_B)], idx_d0)

    def one(off_next, idx_sc, idx_dc, idx_sn, idx_dn):
        pltpu.async_copy(src_hbm.at[pl.ds(off_next, SEG_B)], idx_sn, semi)
        pltpu.async_copy(dst_hbm.at[pl.ds(off_next, SEG_B)], idx_dn, semi)
        pltpu.async_copy(g_hbm.at[idx_sc], rows, semg).wait()
        pltpu.sync_copy(rows, acc.at[idx_dc], add=True)
        _di(semi, idx_sn)
        _di(semi, idx_dn)

    def pair(k2, _):
        off1 = base + (2 * k2 + 1) * SEG_B
        off2 = jnp.minimum(base + (2 * k2 + 2) * SEG_B, last)
        one(off1, idx_s0, idx_d0, idx_s1, idx_d1)
        one(off2, idx_s1, idx_d1, idx_s0, idx_d0)
        return _

    lax.fori_loop(0, (SEG_FULL - 1) // 2, pair, None)
    pltpu.async_copy(g_hbm.at[idx_s0], rows, semg).wait()
    pltpu.sync_copy(rows, acc.at[idx_d0], add=True)

    plsc.subcore_barrier()
    pltpu.sync_copy(acc.at[pl.ds(s * ROWS_T, ROWS_T)],
                    out_hbm.at[c, pl.ds(s * ROWS_T, ROWS_T)])


@functools.cache
def _seg_kernel():
  return pl.kernel(
    _seg_body,
    out_type=jax.ShapeDtypeStruct((NSC, NPAD, HID), jnp.float32),
    mesh=_mesh(),
    scratch_types=[
        pltpu.VMEM((SEG_B,), jnp.int32),
        pltpu.VMEM((SEG_B,), jnp.int32),
        pltpu.VMEM((SEG_B,), jnp.int32),
        pltpu.VMEM((SEG_B,), jnp.int32),
        pltpu.VMEM((SEG_B, HID), jnp.float32),
        pltpu.VMEM_SHARED((NPAD, HID), jnp.float32),
        pltpu.SemaphoreType.DMA,
        pltpu.SemaphoreType.DMA,
    ],
  )


def _seg_call(*args):
    return _seg_kernel()(*args)


def _mask_body(src_hbm, dst_hbm, zeros_hbm, ones_hbm, mask_hbm,
               pall, prel, prel2, zero_v, ones_v, mblk, sems):
    c = lax.axis_index("c")
    s = lax.axis_index("s")

    pltpu.sync_copy(zeros_hbm, zero_v)
    pltpu.sync_copy(ones_hbm, ones_v)
    pltpu.sync_copy(src_hbm.at[pl.ds(s * EDGE_ALL_T, EDGE_ALL_T)],
                    pall.at[pl.ds(0, EDGE_ALL_T)])
    pltpu.sync_copy(dst_hbm.at[pl.ds(s * EDGE_ALL_T, EDGE_ALL_T)],
                    pall.at[pl.ds(PPAD // 2, EDGE_ALL_T)])

    def pos(k, _):
        for u in range(5):
            o = (k * 5 + u) * 16
            sv = pall[pl.ds(o, 16)]
            dv = pall[pl.ds(PPAD // 2 + o, 16)]
            pall[pl.ds(o, 16)] = jnp.where(sv < OUT, dv * OUT + sv, SENT)
            pall[pl.ds(PPAD // 2 + o, 16)] = jnp.where(dv < OUT,
                                                       sv * OUT + dv, SENT)
        return _

    lax.fori_loop(0, EDGE_ALL_T // 80, pos, None)
    for k in range(EDGE_ALL_T // 16, PPAD // 2 // 16):
        pall[pl.ds(k * 16, 16)] = jnp.full((16,), SENT, jnp.int32)
        pall[pl.ds(PPAD // 2 + k * 16, 16)] = jnp.full((16,), SENT, jnp.int32)

    def block(b, _):
        base = (NBLOCK_SC * c + b) * BEL

        for z in range(FTILE // ZCH):
            pltpu.sync_copy(zero_v, mblk.at[pl.ds(s * FTILE + z * ZCH, ZCH)])

        pltpu.sync_copy(zero_v.at[pl.ds(0, 128)],
                        mblk.at[pl.ds(BPAD + s * 128, 128)])

        plsc.subcore_barrier()

        def chunk(kb, dst):
            def sub(j, _):
                for u in range(8):
                    o = (j * 8 + u) * 16
                    pa = pall[pl.ds(kb * SCHUNK + o, 16)]
                    pr = pa - base
                    ok = (pr >= 0) & (pr < BEL)
                    dummy = BPAD + (pa & 2047)
                    dst[pl.ds(o, 16)] = jnp.where(ok, pr, dummy)
                return _

            lax.fori_loop(0, SCHUNK // 128, sub, None)

        def drain():
            pltpu.make_async_copy(ones_hbm, ones_v, sems).wait()

        chunk(0, prel)

        def sb2(k2, _):
            pltpu.async_copy(ones_v, mblk.at[prel], sems)
            chunk(2 * k2 + 1, prel2)
            drain()
            pltpu.async_copy(ones_v, mblk.at[prel2], sems)
            chunk(2 * k2 + 2, prel)
            drain()
            return _

        lax.fori_loop(0, (PPAD // SCHUNK) // 2, sb2, None)
        pltpu.sync_copy(ones_v, mblk.at[prel])
        plsc.subcore_barrier()

        pltpu.sync_copy(mblk.at[pl.ds(s * FTILE, FTILE)],
                        mask_hbm.at[pl.ds(base + s * FTILE, FTILE)])
        return _

    lax.fori_loop(0, NBLOCK_SC, block, None)


@functools.cache
def _mask_kernel():
  return pl.kernel(
    _mask_body,
    out_type=jax.ShapeDtypeStruct((NPAD * OUT,), jnp.float32),
    mesh=_mesh(),
    scratch_types=[
        pltpu.VMEM((PPAD,), jnp.int32),
        pltpu.VMEM((SCHUNK,), jnp.int32),
        pltpu.VMEM((SCHUNK,), jnp.int32),
        pltpu.VMEM((ZCH,), jnp.float32),
        pltpu.VMEM((SCHUNK,), jnp.float32),
        pltpu.VMEM_SHARED((BEL + 2048,), jnp.float32),
        pltpu.SemaphoreType.DMA,
    ],
  )


def _mask_call(*args):
    return _mask_kernel()(*args)


def _deg_body(src_hbm, dst_hbm, zeros_hbm, ones_hbm, deg_hbm,
              idx_d, ones_v, acc):
    c = lax.axis_index("c")
    s = lax.axis_index("s")
    pltpu.sync_copy(ones_hbm, ones_v)
    pltpu.sync_copy(zeros_hbm, acc.at[pl.ds(s * ROWS_T, ROWS_T)])
    plsc.subcore_barrier()

    base = c * EDGE_SC + s * EDGE_T

    def batch(k, _):
        off = base + k * DEG_B
        pltpu.sync_copy(dst_hbm.at[pl.ds(off, DEG_B)], idx_d)
        pltpu.sync_copy(ones_v, acc.at[idx_d], add=True)
        return _

    lax.fori_loop(0, DEG_FULL, batch, None)

    plsc.subcore_barrier()
    pltpu.sync_copy(acc.at[pl.ds(s * ROWS_T, ROWS_T)],
                    deg_hbm.at[c, pl.ds(s * ROWS_T, ROWS_T)])


@functools.cache
def _deg_kernel():
  return pl.kernel(
    _deg_body,
    out_type=jax.ShapeDtypeStruct((NSC, NPAD, HID), jnp.float32),
    mesh=_mesh(),
    scratch_types=[
        pltpu.VMEM((DEG_B,), jnp.int32),
        pltpu.VMEM((DEG_B, HID), jnp.float32),
        pltpu.VMEM_SHARED((NPAD, HID), jnp.float32),
    ],
  )


def _deg_call(*args):
    return _deg_kernel()(*args)


def _dot(a, b):
    return jnp.dot(a, b, preferred_element_type=jnp.float32)


def _tc_a_body(x, sfm, tfm, og, op, wl, wr, g1, r1):
    parts = (x[...], sfm[...], tfm[...], og[...], op[...])
    wlv = wl[...]
    wrv = wr[...]
    g = _dot(parts[0], wlv[0:IN])
    r = _dot(parts[0], wrv[0:IN])
    for k in range(1, 5):
        g = g + _dot(parts[k], wlv[k * IN:(k + 1) * IN])
        r = r + _dot(parts[k], wrv[k * IN:(k + 1) * IN])
    g1[...] = g
    r1[...] = r


def _tc_a(x, sfm, tfm, og, op, wl1t, wr1t):
    bs = pl.BlockSpec((BLK, IN), lambda i: (i, 0))
    ws = pl.BlockSpec((5 * IN, HID), lambda i: (0, 0))
    return pl.pallas_call(
        _tc_a_body,
        grid=(NBLK,),
        in_specs=[bs, bs, bs, bs, bs, ws, ws],
        out_specs=[pl.BlockSpec((BLK, HID), lambda i: (i, 0))] * 2,
        out_shape=[jax.ShapeDtypeStruct((N, HID), jnp.float32)] * 2,
    )(x, sfm, tfm, og, op, wl1t, wr1t)


def _deg_stats(degp):
    deg = degp[0, :, 0] + degp[1, :, 0]
    invdeg = 1.0 / jnp.maximum(deg, 1.0)
    degpos = (deg > 0).astype(jnp.float32)
    return invdeg, degpos


def _layernorm(h, g, b):
    m = jnp.mean(h, axis=-1, keepdims=True)
    v = jnp.var(h, axis=-1, keepdims=True)
    return (h - m) / jnp.sqrt(v + 1e-5) * g + b


def _tc_b_body(s1, degp, r1, tfm, bl1, g1n, b1n, wl2a, wr2a, g2, r2, t):
    i = pl.program_id(0)
    invdeg, _ = _deg_stats(degp[...])
    pre = (s1[0] + s1[1]) * invdeg[:, None] + bl1[...] + r1[...]
    h1 = _layernorm(jax.nn.relu(pre), g1n[...], b1n[...])
    g2[...] = _dot(h1, wl2a[...])
    r2[...] = _dot(h1, wr2a[...])
    flag = jnp.any(tfm[...] != 0, axis=1).astype(jnp.float32)
    tp = _dot(flag[None, :], h1)

    @pl.when(i == 0)
    def _init():
        t[...] = jnp.zeros_like(t)

    t[...] += tp


def _tc_b(s1, degp, r1, tfm, bl1, g1n, b1n, wl2at, wr2at):
    bs = pl.BlockSpec((BLK, HID), lambda i: (i, 0))
    ws = pl.BlockSpec((HID, HID), lambda i: (0, 0))
    vs = pl.BlockSpec((1, HID), lambda i: (0, 0))
    return pl.pallas_call(
        _tc_b_body,
        grid=(NBLK,),
        in_specs=[pl.BlockSpec((NSC, BLK, HID), lambda i: (0, i, 0)),
                  pl.BlockSpec((NSC, BLK, HID), lambda i: (0, i, 0)),
                  bs, bs, vs, vs, vs, ws, ws],
        out_specs=[bs, bs, vs],
        out_shape=[jax.ShapeDtypeStruct((N, HID), jnp.float32),
                   jax.ShapeDtypeStruct((N, HID), jnp.float32),
                   jax.ShapeDtypeStruct((1, HID), jnp.float32)],
    )(s1, degp, r1, tfm, bl1, g1n, b1n, wl2at, wr2at)


def _tc_c_body(s2, degp, r2, t, wl2b, wr2b, bl2, g2n, b2n, wl3, wr3, g3, r3):
    invdeg, degpos = _deg_stats(degp[...])
    tv = t[...]
    tl = _dot(tv, wl2b[...])
    tr = _dot(tv, wr2b[...])
    pre = ((s2[0] + s2[1]) * invdeg[:, None] + degpos[:, None] * tl
           + bl2[...] + r2[...] + tr)
    h2 = _layernorm(jax.nn.relu(pre), g2n[...], b2n[...])
    g3[...] = _dot(h2, wl3[...])
    r3[...] = _dot(h2, wr3[...])


def _tc_c(s2, degp, r2, t, wl2bt, wr2bt, bl2, g2n, b2n, wl3t, wr3t):
    bs = pl.BlockSpec((BLK, HID), lambda i: (i, 0))
    ws = pl.BlockSpec((HID, HID), lambda i: (0, 0))
    vs = pl.BlockSpec((1, HID), lambda i: (0, 0))
    return pl.pallas_call(
        _tc_c_body,
        grid=(NBLK,),
        in_specs=[pl.BlockSpec((NSC, BLK, HID), lambda i: (0, i, 0)),
                  pl.BlockSpec((NSC, BLK, HID), lambda i: (0, i, 0)),
                  bs, vs, ws, ws, vs, vs, vs, ws, ws],
        out_specs=[bs, bs],
        out_shape=[jax.ShapeDtypeStruct((N, HID), jnp.float32)] * 2,
    )(s2, degp, r2, t, wl2bt, wr2bt, bl2, g2n, b2n, wl3t, wr3t)


def _tc_d_body(s3, degp, r3, bl3, fcw, fcb, mask, out):
    i = pl.program_id(0)
    invdeg, _ = _deg_stats(degp[...])
    h3 = jax.nn.relu((s3[0] + s3[1]) * invdeg[:, None] + bl3[...] + r3[...])
    logits = _dot(h3, fcw[...]) + fcb[...]
    rid = i * BLK + lax.broadcasted_iota(jnp.int32, (BLK, OUT), 0)
    cid = lax.broadcasted_iota(jnp.int32, (BLK, OUT), 1)
    maskv = jnp.maximum(mask[...], (rid == cid).astype(jnp.float32))
    mx = jnp.max(logits, axis=1, keepdims=True)
    e = jnp.exp(logits - mx) * maskv
    z = jnp.sum(e, axis=1, keepdims=True)
    out[...] = jnp.where(z > 0, e / jnp.where(z > 0, z, 1.0), 0.0)


def _tc_d(s3, degp, r3, bl3, fcwt, fcb, maskm):
    bs = pl.BlockSpec((BLK, HID), lambda i: (i, 0))
    return pl.pallas_call(
        _tc_d_body,
        grid=(NBLK,),
        in_specs=[pl.BlockSpec((NSC, BLK, HID), lambda i: (0, i, 0)),
                  pl.BlockSpec((NSC, BLK, HID), lambda i: (0, i, 0)),
                  bs,
                  pl.BlockSpec((1, HID), lambda i: (0, 0)),
                  pl.BlockSpec((HID, OUT), lambda i: (0, 0)),
                  pl.BlockSpec((1, OUT), lambda i: (0, 0)),
                  pl.BlockSpec((BLK, OUT), lambda i: (i, 0))],
        out_specs=pl.BlockSpec((BLK, OUT), lambda i: (i, 0)),
        out_shape=jax.ShapeDtypeStruct((N, OUT), jnp.float32),
    )(s3, degp, r3, bl3, fcwt, fcb, maskm)


def kernel(x, start_feature_masked, target_feature_masked, other_goals,
           other_pos, edge_index,
           conv1_Wl, conv1_bl, conv1_Wr, conv2_Wl, conv2_bl, conv2_Wr,
           conv3_Wl, conv3_bl, conv3_Wr, fc_W, fc_b,
           ln1_g, ln1_b, ln2_g, ln2_b):
    src = edge_index[0]
    dst = edge_index[1]

    wl1t = conv1_Wl.T
    wr1t = conv1_Wr.T
    wl2at = conv2_Wl[:, :HID].T
    wl2bt = conv2_Wl[:, HID:].T
    wr2at = conv2_Wr[:, :HID].T
    wr2bt = conv2_Wr[:, HID:].T
    wl3t = conv3_Wl.T
    wr3t = conv3_Wr.T
    fcwt = fc_W.T

    bl1 = conv1_bl.reshape(1, HID)
    bl2 = conv2_bl.reshape(1, HID)
    bl3 = conv3_bl.reshape(1, HID)
    fcb = fc_b.reshape(1, OUT)
    g1n = ln1_g.reshape(1, HID)
    b1n = ln1_b.reshape(1, HID)
    g2n = ln2_g.reshape(1, HID)
    b2n = ln2_b.reshape(1, HID)

    zeros_a = jnp.zeros((ROWS_T, HID), jnp.float32)
    zeros_m = jnp.zeros((ZCH,), jnp.float32)
    ones_d = jnp.ones((DEG_B, HID), jnp.float32)
    ones_m = jnp.ones((SCHUNK,), jnp.float32)

    mask_flat = _mask_call(src, dst, zeros_m, ones_m)
    degp = _deg_call(src, dst, zeros_a, ones_d)
    maskm = mask_flat.reshape(NPAD, OUT)

    g1, r1 = _tc_a(x, start_feature_masked, target_feature_masked,
                   other_goals, other_pos, wl1t, wr1t)
    s1 = _seg_call(g1, src, dst, zeros_a)
    g2, r2, t = _tc_b(s1, degp, r1, target_feature_masked, bl1, g1n, b1n,
                      wl2at, wr2at)
    s2 = _seg_call(g2, src, dst, zeros_a)
    g3, r3 = _tc_c(s2, degp, r2, t, wl2bt, wr2bt, bl2, g2n, b2n, wl3t, wr3t)
    s3 = _seg_call(g3, src, dst, zeros_a)
    return _tc_d(s3, degp, r3, bl3, fcwt, fcb, maskm)
